# baseline (speedup 1.0000x reference)
import jax
import jax.numpy as jnp
from jax import lax
from jax.experimental import pallas as pl
from jax.experimental.pallas import tpu as pltpu

B, H, D = 8, 16, 128
CAP = 2048
CHUNK = 16
CAPT = CAP // CHUNK


def _body(pos_ref,
          kval_ref, vval_ref,
          kt_ref,
          kc_hbm, vc_hbm,
          kt_out_ref,
          ko_hbm, vo_hbm,
          bulk_sem,
          row_sem):
    i = pl.program_id(0)
    n = pl.num_programs(0)
    pos = pos_ref[0]

    @pl.when(i == 0)
    def _start_bulk():
        for b in range(B):
            pltpu.make_async_copy(kc_hbm.at[b], ko_hbm.at[b],
                                  bulk_sem.at[0, b]).start()
            pltpu.make_async_copy(vc_hbm.at[b], vo_hbm.at[b],
                                  bulk_sem.at[1, b]).start()

    bh_b = i // H
    bh_h = lax.rem(i, H)
    kt = kt_ref[0, 0]
    t = kt.T
    kv = kval_ref[pl.ds(bh_b, 1), pl.ds(bh_h, 1), 0, :].reshape(1, D)
    kv2 = jnp.concatenate([kv, kv], axis=-1)
    col = lax.broadcasted_iota(jnp.int32, (CAPT, 2 * D), 1)
    row = lax.broadcasted_iota(jnp.int32, (CAPT, 2 * D), 0)
    merged = jnp.where(col < D, jnp.minimum(t, kv2), jnp.maximum(t, kv2))
    chunk_idx = pos // CHUNK
    kt_out_ref[0, 0] = jnp.where(row == chunk_idx, merged, t)

    @pl.when(i == n - 1)
    def _finish():
        for b in range(B):
            pltpu.make_async_copy(kc_hbm.at[b], ko_hbm.at[b],
                                  bulk_sem.at[0, b]).wait()
            pltpu.make_async_copy(vc_hbm.at[b], vo_hbm.at[b],
                                  bulk_sem.at[1, b]).wait()
        krow = pltpu.make_async_copy(
            kval_ref, ko_hbm.at[:, :, pl.ds(pos, 1), :], row_sem.at[0])
        vrow = pltpu.make_async_copy(
            vval_ref, vo_hbm.at[:, :, pl.ds(pos, 1), :], row_sem.at[1])
        krow.start()
        vrow.start()
        krow.wait()
        vrow.wait()


def kernel(input_pos, q, k_val, v_val, k_cache, kt_cache, v_cache):
    del q
    pos32 = input_pos.astype(jnp.int32)

    grid = (B * H,)
    kt_out, k_out, v_out = pl.pallas_call(
        _body,
        grid=grid,
        in_specs=[
            pl.BlockSpec(memory_space=pltpu.SMEM),
            pl.BlockSpec((B, H, 1, D), lambda i: (0, 0, 0, 0)),
            pl.BlockSpec((B, H, 1, D), lambda i: (0, 0, 0, 0)),
            pl.BlockSpec((1, 1, 2 * D, CAPT),
                         lambda i: (i // H, i % H, 0, 0)),
            pl.BlockSpec(memory_space=pltpu.HBM),
            pl.BlockSpec(memory_space=pltpu.HBM),
        ],
        out_specs=[
            pl.BlockSpec((1, 1, CAPT, 2 * D),
                         lambda i: (i // H, i % H, 0, 0)),
            pl.BlockSpec(memory_space=pltpu.HBM),
            pl.BlockSpec(memory_space=pltpu.HBM),
        ],
        out_shape=[
            jax.ShapeDtypeStruct((B, H, CAPT, 2 * D), jnp.float32),
            jax.ShapeDtypeStruct((B, H, CAP, D), jnp.float32),
            jax.ShapeDtypeStruct((B, H, CAP, D), jnp.float32),
        ],
        scratch_shapes=[
            pltpu.SemaphoreType.DMA((2, B)),
            pltpu.SemaphoreType.DMA((2,)),
        ],
        compiler_params=pltpu.CompilerParams(
            dimension_semantics=("arbitrary",),
        ),
    )(pos32, k_val, v_val, kt_cache, k_cache, v_cache)
    return (kt_out, k_out, v_out)


# speedup vs baseline: 20.8242x; 20.8242x over previous
import jax
import jax.numpy as jnp
from jax import lax
from jax.experimental import pallas as pl
from jax.experimental.pallas import tpu as pltpu

B, H, D = 8, 16, 128
CAP = 2048
CHUNK = 16
CAPT = CAP // CHUNK
ROWS = 512
NJ = CAP // ROWS


def _body(pos_ref,
          kval_ref, vval_ref,
          kt_ref,
          kc_ref, vc_ref,
          kt_out_ref,
          ko_ref, vo_ref):
    i = pl.program_id(0)
    j = pl.program_id(1)
    pos = pos_ref[0]
    bh_b = i // H
    bh_h = lax.rem(i, H)

    kv = kval_ref[pl.ds(bh_b, 1), pl.ds(bh_h, 1), 0, :].reshape(1, D)
    vv = vval_ref[pl.ds(bh_b, 1), pl.ds(bh_h, 1), 0, :].reshape(1, D)

    r = j * ROWS + lax.broadcasted_iota(jnp.int32, (ROWS, D), 0)
    hit = r == pos
    ko_ref[0, 0] = jnp.where(hit, kv, kc_ref[0, 0])
    vo_ref[0, 0] = jnp.where(hit, vv, vc_ref[0, 0])

    kt = kt_ref[0, 0]
    t = kt.T
    kv2 = jnp.concatenate([kv, kv], axis=-1)
    col = lax.broadcasted_iota(jnp.int32, (CAPT, 2 * D), 1)
    row = lax.broadcasted_iota(jnp.int32, (CAPT, 2 * D), 0)
    merged = jnp.where(col < D, jnp.minimum(t, kv2), jnp.maximum(t, kv2))
    chunk_idx = pos // CHUNK
    kt_out_ref[0, 0] = jnp.where(row == chunk_idx, merged, t)


def kernel(input_pos, q, k_val, v_val, k_cache, kt_cache, v_cache):
    del q
    pos32 = input_pos.astype(jnp.int32)

    grid = (B * H, NJ)
    kt_out, k_out, v_out = pl.pallas_call(
        _body,
        grid=grid,
        in_specs=[
            pl.BlockSpec(memory_space=pltpu.SMEM),
            pl.BlockSpec((B, H, 1, D), lambda i, j: (0, 0, 0, 0)),
            pl.BlockSpec((B, H, 1, D), lambda i, j: (0, 0, 0, 0)),
            pl.BlockSpec((1, 1, 2 * D, CAPT),
                         lambda i, j: (i // H, i % H, 0, 0)),
            pl.BlockSpec((1, 1, ROWS, D),
                         lambda i, j: (i // H, i % H, j, 0)),
            pl.BlockSpec((1, 1, ROWS, D),
                         lambda i, j: (i // H, i % H, j, 0)),
        ],
        out_specs=[
            pl.BlockSpec((1, 1, CAPT, 2 * D),
                         lambda i, j: (i // H, i % H, 0, 0)),
            pl.BlockSpec((1, 1, ROWS, D),
                         lambda i, j: (i // H, i % H, j, 0)),
            pl.BlockSpec((1, 1, ROWS, D),
                         lambda i, j: (i // H, i % H, j, 0)),
        ],
        out_shape=[
            jax.ShapeDtypeStruct((B, H, CAPT, 2 * D), jnp.float32),
            jax.ShapeDtypeStruct((B, H, CAP, D), jnp.float32),
            jax.ShapeDtypeStruct((B, H, CAP, D), jnp.float32),
        ],
        compiler_params=pltpu.CompilerParams(
            dimension_semantics=("parallel", "parallel"),
        ),
    )(pos32, k_val, v_val, kt_cache, k_cache, v_cache)
    return (kt_out, k_out, v_out)


# speedup vs baseline: 27.8652x; 1.3381x over previous
import jax
import jax.numpy as jnp
from jax import lax
from jax.experimental import pallas as pl
from jax.experimental.pallas import tpu as pltpu

B, H, D = 8, 16, 128
CAP = 2048
CHUNK = 16
CAPT = CAP // CHUNK
ROWS = 1024
NJ = CAP // ROWS


def _body(pos_ref,
          kval_ref, vval_ref,
          kt_ref,
          kc_ref, vc_ref,
          kt_out_ref,
          ko_ref, vo_ref):
    i = pl.program_id(0)
    j = pl.program_id(1)
    pos = pos_ref[0]
    bh_b = i // H
    bh_h = lax.rem(i, H)

    kv = kval_ref[pl.ds(bh_b, 1), pl.ds(bh_h, 1), 0, :].reshape(1, D)
    vv = vval_ref[pl.ds(bh_b, 1), pl.ds(bh_h, 1), 0, :].reshape(1, D)

    r = j * ROWS + lax.broadcasted_iota(jnp.int32, (ROWS, D), 0)
    hit = r == pos
    ko_ref[0, 0] = jnp.where(hit, kv, kc_ref[0, 0])
    vo_ref[0, 0] = jnp.where(hit, vv, vc_ref[0, 0])

    @pl.when(j == 0)
    def _kt():
        kt = kt_ref[0, 0]
        t = kt.T
        kv2 = jnp.concatenate([kv, kv], axis=-1)
        col = lax.broadcasted_iota(jnp.int32, (CAPT, 2 * D), 1)
        row = lax.broadcasted_iota(jnp.int32, (CAPT, 2 * D), 0)
        merged = jnp.where(col < D, jnp.minimum(t, kv2), jnp.maximum(t, kv2))
        chunk_idx = pos // CHUNK
        kt_out_ref[0, 0] = jnp.where(row == chunk_idx, merged, t)


def kernel(input_pos, q, k_val, v_val, k_cache, kt_cache, v_cache):
    del q
    pos32 = input_pos.astype(jnp.int32)

    grid = (B * H, NJ)
    kt_out, k_out, v_out = pl.pallas_call(
        _body,
        grid=grid,
        in_specs=[
            pl.BlockSpec(memory_space=pltpu.SMEM),
            pl.BlockSpec((B, H, 1, D), lambda i, j: (0, 0, 0, 0)),
            pl.BlockSpec((B, H, 1, D), lambda i, j: (0, 0, 0, 0)),
            pl.BlockSpec((1, 1, 2 * D, CAPT),
                         lambda i, j: (i // H, i % H, 0, 0)),
            pl.BlockSpec((1, 1, ROWS, D),
                         lambda i, j: (i // H, i % H, j, 0)),
            pl.BlockSpec((1, 1, ROWS, D),
                         lambda i, j: (i // H, i % H, j, 0)),
        ],
        out_specs=[
            pl.BlockSpec((1, 1, CAPT, 2 * D),
                         lambda i, j: (i // H, i % H, 0, 0)),
            pl.BlockSpec((1, 1, ROWS, D),
                         lambda i, j: (i // H, i % H, j, 0)),
            pl.BlockSpec((1, 1, ROWS, D),
                         lambda i, j: (i // H, i % H, j, 0)),
        ],
        out_shape=[
            jax.ShapeDtypeStruct((B, H, CAPT, 2 * D), jnp.float32),
            jax.ShapeDtypeStruct((B, H, CAP, D), jnp.float32),
            jax.ShapeDtypeStruct((B, H, CAP, D), jnp.float32),
        ],
        compiler_params=pltpu.CompilerParams(
            dimension_semantics=("parallel", "parallel"),
        ),
    )(pos32, k_val, v_val, kt_cache, k_cache, v_cache)
    return (kt_out, k_out, v_out)


# speedup vs baseline: 28.9753x; 1.0398x over previous
import jax
import jax.numpy as jnp
from jax import lax
from jax.experimental import pallas as pl
from jax.experimental.pallas import tpu as pltpu
from jax.experimental.pallas import tpu_sc as plsc

B, H, D = 8, 16, 128
CAP = 2048
CHUNK = 16
CAPT = CAP // CHUNK
ROWS = 1024
NJ = CAP // ROWS

NC, NS = 2, 16
NW = NC * NS
TOTAL_ROWS = B * H * CAP
RPW = TOTAL_ROWS // NW
CH = 256
NCHUNK = RPW // CH
NBUF = 3


def _sc_body(k_hbm, kval_hbm, pos_hbm, ko_hbm,
             buf, ld_sem, st_sem, kval_v, pos_v, idx_v, rs_sem):
    c = lax.axis_index("c")
    s = lax.axis_index("s")
    wid = c * NS + s
    base = wid * RPW

    for b in range(NBUF):
        pltpu.make_async_copy(k_hbm.at[pl.ds(base + b * CH, CH)],
                              buf.at[b], ld_sem.at[b]).start()
    for g in range(NCHUNK):
        b = g % NBUF
        pltpu.make_async_copy(k_hbm.at[pl.ds(base + g * CH, CH)],
                              buf.at[b], ld_sem.at[b]).wait()
        st = pltpu.make_async_copy(buf.at[b],
                                   ko_hbm.at[pl.ds(base + g * CH, CH)],
                                   st_sem.at[b])
        st.start()
        if g + NBUF < NCHUNK:
            st.wait()
            pltpu.make_async_copy(k_hbm.at[pl.ds(base + (g + NBUF) * CH, CH)],
                                  buf.at[b], ld_sem.at[b]).start()
    for g in range(NCHUNK - NBUF, NCHUNK):
        b = g % NBUF
        pltpu.make_async_copy(buf.at[b],
                              ko_hbm.at[pl.ds(base + g * CH, CH)],
                              st_sem.at[b]).wait()

    pltpu.make_async_copy(pos_hbm, pos_v, rs_sem).start()
    pltpu.make_async_copy(pos_hbm, pos_v, rs_sem).wait()
    for r in range(4):
        cp = pltpu.make_async_copy(kval_hbm.at[pl.ds(wid * 4, 4)],
                                   kval_v.at[pl.ds(4 * r, 4)], rs_sem)
        cp.start()
        cp.wait()
    lane = lax.iota(jnp.int32, 16)
    bh = wid * 4 + lax.rem(lane, 4)
    idx_v[...] = bh * CAP + pos_v[...]
    sc = pltpu.make_async_copy(kval_v, ko_hbm.at[idx_v], rs_sem)
    sc.start()
    sc.wait()


_sc_copy = pl.kernel(
    _sc_body,
    out_type=jax.ShapeDtypeStruct((TOTAL_ROWS, D), jnp.float32),
    mesh=plsc.VectorSubcoreMesh(core_axis_name="c", subcore_axis_name="s",
                                num_cores=NC, num_subcores=NS),
    scratch_types=[
        pltpu.VMEM((NBUF, CH, D), jnp.float32),
        pltpu.SemaphoreType.DMA((NBUF,)),
        pltpu.SemaphoreType.DMA((NBUF,)),
        pltpu.VMEM((16, D), jnp.float32),
        pltpu.VMEM((16,), jnp.int32),
        pltpu.VMEM((16,), jnp.int32),
        pltpu.SemaphoreType.DMA,
    ],
)


def _tc_body(pos_ref,
             kval_ref, vval_ref,
             kt_ref,
             vc_ref,
             kt_out_ref,
             vo_ref):
    i = pl.program_id(0)
    j = pl.program_id(1)
    pos = pos_ref[0]
    bh_b = i // H
    bh_h = lax.rem(i, H)

    kv = kval_ref[pl.ds(bh_b, 1), pl.ds(bh_h, 1), 0, :].reshape(1, D)
    vv = vval_ref[pl.ds(bh_b, 1), pl.ds(bh_h, 1), 0, :].reshape(1, D)

    r = j * ROWS + lax.broadcasted_iota(jnp.int32, (ROWS, D), 0)
    vo_ref[0, 0] = jnp.where(r == pos, vv, vc_ref[0, 0])

    @pl.when(j == 0)
    def _kt():
        kt = kt_ref[0, 0]
        t = kt.T
        kv2 = jnp.concatenate([kv, kv], axis=-1)
        col = lax.broadcasted_iota(jnp.int32, (CAPT, 2 * D), 1)
        row = lax.broadcasted_iota(jnp.int32, (CAPT, 2 * D), 0)
        merged = jnp.where(col < D, jnp.minimum(t, kv2), jnp.maximum(t, kv2))
        chunk_idx = pos // CHUNK
        kt_out_ref[0, 0] = jnp.where(row == chunk_idx, merged, t)


def kernel(input_pos, q, k_val, v_val, k_cache, kt_cache, v_cache):
    del q
    pos32 = input_pos.astype(jnp.int32)
    pos16 = jnp.broadcast_to(pos32, (16,))

    k_out = _sc_copy(k_cache.reshape(TOTAL_ROWS, D),
                     k_val.reshape(B * H, D), pos16)

    grid = (B * H, NJ)
    kt_out, v_out = pl.pallas_call(
        _tc_body,
        grid=grid,
        in_specs=[
            pl.BlockSpec(memory_space=pltpu.SMEM),
            pl.BlockSpec((B, H, 1, D), lambda i, j: (0, 0, 0, 0)),
            pl.BlockSpec((B, H, 1, D), lambda i, j: (0, 0, 0, 0)),
            pl.BlockSpec((1, 1, 2 * D, CAPT),
                         lambda i, j: (i // H, i % H, 0, 0)),
            pl.BlockSpec((1, 1, ROWS, D),
                         lambda i, j: (i // H, i % H, j, 0)),
        ],
        out_specs=[
            pl.BlockSpec((1, 1, CAPT, 2 * D),
                         lambda i, j: (i // H, i % H, 0, 0)),
            pl.BlockSpec((1, 1, ROWS, D),
                         lambda i, j: (i // H, i % H, j, 0)),
        ],
        out_shape=[
            jax.ShapeDtypeStruct((B, H, CAPT, 2 * D), jnp.float32),
            jax.ShapeDtypeStruct((B, H, CAP, D), jnp.float32),
        ],
        compiler_params=pltpu.CompilerParams(
            dimension_semantics=("parallel", "parallel"),
        ),
    )(pos32, k_val, v_val, kt_cache, v_cache)
    return (kt_out, k_out.reshape(B, H, CAP, D), v_out)


# speedup vs baseline: 36.0663x; 1.2447x over previous
import jax
import jax.numpy as jnp
from jax import lax
from jax.experimental import pallas as pl
from jax.experimental.pallas import tpu as pltpu
from jax.experimental.pallas import tpu_sc as plsc

B, H, D = 8, 16, 128
CAP = 2048
CHUNK = 16
CAPT = CAP // CHUNK
ROWS = 1024
NJ = CAP // ROWS

NC, NS = 2, 16
NW = NC * NS
TOTAL_ROWS = B * H * CAP
RPW = TOTAL_ROWS // NW
CH = 256
NCHUNK = RPW // CH
NBUF = 3


def _sc_body(k_hbm, kval_hbm, pos_hbm, ko_hbm,
             buf, ld_sem, st_sem, kval_v, pos_v, idx_v, rs_sem):
    c = lax.axis_index("c")
    s = lax.axis_index("s")
    wid = c * NS + s
    base = wid * RPW

    for b in range(NBUF):
        pltpu.make_async_copy(k_hbm.at[pl.ds(base + b * CH, CH)],
                              buf.at[b], ld_sem.at[b]).start()
    for g in range(NCHUNK):
        b = g % NBUF
        pltpu.make_async_copy(k_hbm.at[pl.ds(base + g * CH, CH)],
                              buf.at[b], ld_sem.at[b]).wait()
        st = pltpu.make_async_copy(buf.at[b],
                                   ko_hbm.at[pl.ds(base + g * CH, CH)],
                                   st_sem.at[b])
        st.start()
        if g + NBUF < NCHUNK:
            st.wait()
            pltpu.make_async_copy(k_hbm.at[pl.ds(base + (g + NBUF) * CH, CH)],
                                  buf.at[b], ld_sem.at[b]).start()
    for g in range(NCHUNK - NBUF, NCHUNK):
        b = g % NBUF
        pltpu.make_async_copy(buf.at[b],
                              ko_hbm.at[pl.ds(base + g * CH, CH)],
                              st_sem.at[b]).wait()

    pltpu.make_async_copy(pos_hbm, pos_v, rs_sem).start()
    pltpu.make_async_copy(pos_hbm, pos_v, rs_sem).wait()
    for r in range(4):
        cp = pltpu.make_async_copy(kval_hbm.at[pl.ds(wid * 4, 4)],
                                   kval_v.at[pl.ds(4 * r, 4)], rs_sem)
        cp.start()
        cp.wait()
    lane = lax.iota(jnp.int32, 16)
    bh = wid * 4 + lax.rem(lane, 4)
    idx_v[...] = bh * CAP + pos_v[...]
    sc = pltpu.make_async_copy(kval_v, ko_hbm.at[idx_v], rs_sem)
    sc.start()
    sc.wait()


_sc_copy = pl.kernel(
    _sc_body,
    out_type=jax.ShapeDtypeStruct((TOTAL_ROWS, D), jnp.float32),
    mesh=plsc.VectorSubcoreMesh(core_axis_name="c", subcore_axis_name="s",
                                num_cores=NC, num_subcores=NS),
    scratch_types=[
        pltpu.VMEM((NBUF, CH, D), jnp.float32),
        pltpu.SemaphoreType.DMA((NBUF,)),
        pltpu.SemaphoreType.DMA((NBUF,)),
        pltpu.VMEM((16, D), jnp.float32),
        pltpu.VMEM((16,), jnp.int32),
        pltpu.VMEM((16,), jnp.int32),
        pltpu.SemaphoreType.DMA,
    ],
)


def _tc_v_body(pos_ref,
               vval_ref,
               vc_ref,
               vo_ref):
    i = pl.program_id(0)
    pos = pos_ref[0]
    bh_b = i // H
    bh_h = lax.rem(i, H)
    vv = vval_ref[pl.ds(bh_b, 1), pl.ds(bh_h, 1), 0, :].reshape(1, D)
    r = lax.broadcasted_iota(jnp.int32, (CAP, D), 0)
    vo_ref[0, 0] = jnp.where(r == pos, vv, vc_ref[0, 0])


def _tc_kt_body(pos_ref,
                kval_ref,
                kt_ref,
                kt_out_ref):
    pos = pos_ref[0]
    kt = kt_ref[0]
    t = jnp.swapaxes(kt, -1, -2)
    kv = kval_ref[0, :, 0, :]
    kv2 = jnp.concatenate([kv, kv], axis=-1)[:, None, :]
    col = lax.broadcasted_iota(jnp.int32, (NKT, CAPT, 2 * D), 2)
    row = lax.broadcasted_iota(jnp.int32, (NKT, CAPT, 2 * D), 1)
    merged = jnp.where(col < D, jnp.minimum(t, kv2), jnp.maximum(t, kv2))
    chunk_idx = pos // CHUNK
    kt_out_ref[0] = jnp.where(row == chunk_idx, merged, t)


NKT = 16


def kernel(input_pos, q, k_val, v_val, k_cache, kt_cache, v_cache):
    del q
    pos32 = input_pos.astype(jnp.int32)
    pos16 = jnp.broadcast_to(pos32, (16,))

    k_out = _sc_copy(k_cache.reshape(TOTAL_ROWS, D),
                     k_val.reshape(B * H, D), pos16)

    v_out = pl.pallas_call(
        _tc_v_body,
        grid=(B * H,),
        in_specs=[
            pl.BlockSpec(memory_space=pltpu.SMEM),
            pl.BlockSpec((B, H, 1, D), lambda i: (0, 0, 0, 0)),
            pl.BlockSpec((1, 1, CAP, D),
                         lambda i: (i // H, i % H, 0, 0)),
        ],
        out_specs=pl.BlockSpec((1, 1, CAP, D),
                               lambda i: (i // H, i % H, 0, 0)),
        out_shape=jax.ShapeDtypeStruct((B, H, CAP, D), jnp.float32),
        compiler_params=pltpu.CompilerParams(
            dimension_semantics=("parallel",),
        ),
    )(pos32, v_val, v_cache)

    kt_out = pl.pallas_call(
        _tc_kt_body,
        grid=(B,),
        in_specs=[
            pl.BlockSpec(memory_space=pltpu.SMEM),
            pl.BlockSpec((1, NKT, 1, D), lambda i: (i, 0, 0, 0)),
            pl.BlockSpec((1, NKT, 2 * D, CAPT),
                         lambda i: (i, 0, 0, 0)),
        ],
        out_specs=pl.BlockSpec((1, NKT, CAPT, 2 * D),
                               lambda i: (i, 0, 0, 0)),
        out_shape=jax.ShapeDtypeStruct((B, H, CAPT, 2 * D), jnp.float32),
        compiler_params=pltpu.CompilerParams(
            dimension_semantics=("parallel",),
        ),
    )(pos32, k_val, kt_cache)

    return (kt_out, k_out.reshape(B, H, CAP, D), v_out)
